# baseline (speedup 1.0000x reference)
import jax
import jax.numpy as jnp
from jax import lax
from jax.experimental import pallas as pl
from jax.experimental.pallas import tpu as pltpu
from jax.experimental.pallas import tpu_sc as plsc

N = 10000
D = 128
E = 320000

NC = 2
NS = 16
NW = NC * NS
HN = N // 2

CHUNK = 128
ETW = 10240
BROWS = ETW // CHUNK
BSLAB = BROWS // 2
CAPCH = 128
OUTCH = 88
CAPW = OUTCH * CHUNK + CHUNK
ASLAB = 64

TPRH = 320
ACC_H = NS * TPRH
SPARE = ACC_H - HN
HLD = 320

ROW_BLK = 1000


def _sc_mesh():
  return plsc.VectorSubcoreMesh(core_axis_name="c", subcore_axis_name="s")


def _iota16():
  return jax.lax.iota(jnp.int32, 16)


def _bin_body(src_hbm, dst_hbm, bsrc_hbm, bdst_hbm, ncnt_hbm,
              sslab, dslab, st0, st1, st2, st3, ubs, ubd, cntb):
  cid = lax.axis_index("c")
  sid = lax.axis_index("s")
  wid = cid * NS + sid
  stages = (st0, st1, st2, st3)
  iot = _iota16()

  one16 = jnp.ones((16,), jnp.int32)

  def _row(r, cur):
    curs = list(cur)
    for c in range(CHUNK // 16):
      srcv = sslab[r, pl.ds(c * 16, 16)]
      dstv = dslab[r, pl.ds(c * 16, 16)]
      sh = jnp.where(srcv >= HN, 1, 0)
      dh = jnp.where(dstv >= HN, 1, 0)
      cross = sh != dh
      srcx = jnp.where(cross, srcv, srcv - sh * HN)
      dstr = dstv - dh * HN
      pack = srcx * 8192 + dstr
      q = dh * 2 + jnp.where(cross, 1, 0)
      for b in range(4):
        m = q == b
        s = jnp.where(m, one16, 0)
        for d in (1, 2, 4, 8):
          s = s + jnp.where(iot >= d, s[jnp.maximum(iot - d, 0)], 0)
        plsc.store_scatter(stages[b], [curs[b] + s - 1], pack, mask=m)
        curs[b] = curs[b] + plsc.all_reduce_population_count(m)
    return tuple(curs)

  zero16 = jnp.zeros((16,), jnp.int32)
  cur = (zero16, zero16, zero16, zero16)
  for stage in range(2):
    pltpu.sync_copy(src_hbm.at[wid, pl.ds(stage * BSLAB, BSLAB)], sslab)
    pltpu.sync_copy(dst_hbm.at[wid, pl.ds(stage * BSLAB, BSLAB)], dslab)
    cur = lax.fori_loop(0, BSLAB, _row, cur)

  dummy = 5000 + iot
  for b in range(4):
    for k in range(CHUNK // 16):
      plsc.store_scatter(stages[b], [cur[b] + k * 16 + iot], dummy)
  for k in range(CHUNK // 16):
    cntb[pl.ds(k * 16, 16)] = zero16
  for b in range(4):
    nch = jnp.right_shift(cur[b] + CHUNK - 1, 7)
    cntb[pl.ds(b * 16, 16)] = nch
  pltpu.sync_copy(cntb, ncnt_hbm.at[wid])

  def _unp(stage):
    def body(j, carry):
      for k in range(CHUNK // 16):
        pk = stage[pl.ds(j * CHUNK + k * 16, 16)]
        ubs[j, pl.ds(k * 16, 16)] = jnp.right_shift(pk, 13)
        ubd[j, pl.ds(k * 16, 16)] = jnp.bitwise_and(pk, 8191)
      return carry
    return body
  for b in range(4):
    lax.fori_loop(0, OUTCH, _unp(stages[b]), 0)
    pltpu.sync_copy(ubs, bsrc_hbm.at[wid, b, pl.ds(0, OUTCH)])
    pltpu.sync_copy(ubd, bdst_hbm.at[wid, b, pl.ds(0, OUTCH)])


def _sc_bin(src_b, dst_b):
  kern = pl.kernel(
      _bin_body,
      out_type=[
          jax.ShapeDtypeStruct((NW, 4, CAPCH, CHUNK), jnp.int32),
          jax.ShapeDtypeStruct((NW, 4, CAPCH, CHUNK), jnp.int32),
          jax.ShapeDtypeStruct((NW, CHUNK), jnp.int32),
      ],
      mesh=_sc_mesh(),
      compiler_params=pltpu.CompilerParams(needs_layout_passes=False),
      scratch_types=[
          pltpu.VMEM((BSLAB, CHUNK), jnp.int32),
          pltpu.VMEM((BSLAB, CHUNK), jnp.int32),
          pltpu.VMEM((CAPW,), jnp.int32),
          pltpu.VMEM((CAPW,), jnp.int32),
          pltpu.VMEM((CAPW,), jnp.int32),
          pltpu.VMEM((CAPW,), jnp.int32),
          pltpu.VMEM((OUTCH, CHUNK), jnp.int32),
          pltpu.VMEM((OUTCH, CHUNK), jnp.int32),
          pltpu.VMEM((CHUNK,), jnp.int32),
      ],
  )
  return kern(src_b, dst_b)


def _agg_body(h_hbm, bsrc_hbm, bdst_hbm, ncnt_hbm, out_hbm,
              src_v, dst_v, rows_a, rows_b, cnt_s, ncnt_sh, h_sh, acc_sh,
              sem_a, sem_b):
  cid = lax.axis_index("c")
  sid = lax.axis_index("s")
  iot = _iota16()

  @pl.when(sid == 0)
  def _():
    pltpu.sync_copy(ncnt_hbm, ncnt_sh)

  @pl.when(sid < NS - 1)
  def _():
    pltpu.sync_copy(h_hbm.at[pl.ds(cid * HN + sid * HLD, HLD)],
                    h_sh.at[pl.ds(sid * HLD, HLD)])

  @pl.when(sid == NS - 1)
  def _():
    pltpu.sync_copy(h_hbm.at[pl.ds(cid * HN + (NS - 1) * HLD, HN - (NS - 1) * HLD)],
                    h_sh.at[pl.ds((NS - 1) * HLD, HN - (NS - 1) * HLD)])

  def _zrow(i, carry):
    for j in range(D // 16):
      rows_a[i, pl.ds(j * 16, 16)] = jnp.zeros((16,), jnp.float32)
    return carry
  lax.fori_loop(0, CHUNK, _zrow, 0)
  zbase = sid * TPRH
  pltpu.sync_copy(rows_a, acc_sh.at[pl.ds(zbase, CHUNK)])
  pltpu.sync_copy(rows_a, acc_sh.at[pl.ds(zbase + CHUNK, CHUNK)])
  pltpu.sync_copy(rows_a.at[pl.ds(0, TPRH - 2 * CHUNK)],
                  acc_sh.at[pl.ds(zbase + 2 * CHUNK, TPRH - 2 * CHUNK)])
  plsc.subcore_barrier()

  for g_i in range(2):
    g = sid + NS * g_i
    pltpu.sync_copy(ncnt_sh.at[g, pl.ds(0, 64)], cnt_s)
    for bt in range(2):
      b = cid * 2 + bt
      nch = cnt_s[b * 16]
      nst = (nch + ASLAB - 1) // ASLAB

      if bt == 0:
        def _start(j, buf, sem):
          pltpu.async_copy(h_sh.at[src_v.at[j]], buf, sem)

        def _wait(j, buf, sem):
          pltpu.make_async_copy(h_sh.at[src_v.at[j]], buf, sem).wait()
      else:
        def _start(j, buf, sem):
          pltpu.async_copy(h_hbm.at[src_v.at[j]], buf, sem)

        def _wait(j, buf, sem):
          pltpu.make_async_copy(h_hbm.at[src_v.at[j]], buf, sem).wait()

      def _scat(j, buf):
        pltpu.sync_copy(buf, acc_sh.at[dst_v.at[j]], add=True)

      def _stage(st, carry):
        pltpu.sync_copy(bsrc_hbm.at[g, b, pl.ds(st * ASLAB, ASLAB)], src_v)
        pltpu.sync_copy(bdst_hbm.at[g, b, pl.ds(st * ASLAB, ASLAB)], dst_v)
        cnt_here = jnp.minimum(ASLAB, nch - st * ASLAB)
        _start(0, rows_a, sem_a)

        def _inner(j, c2):
          even = (j % 2) == 0
          nxt = j + 1 < cnt_here

          @pl.when(jnp.logical_and(nxt, even))
          def _():
            _start(j + 1, rows_b, sem_b)

          @pl.when(jnp.logical_and(nxt, jnp.logical_not(even)))
          def _():
            _start(j + 1, rows_a, sem_a)

          @pl.when(even)
          def _():
            _wait(j, rows_a, sem_a)
            _scat(j, rows_a)

          @pl.when(jnp.logical_not(even))
          def _():
            _wait(j, rows_b, sem_b)
            _scat(j, rows_b)
          return c2

        lax.fori_loop(0, cnt_here, _inner, 0)
        return carry

      lax.fori_loop(0, nst, _stage, 0)
  plsc.subcore_barrier()

  pltpu.sync_copy(acc_sh.at[pl.ds(sid * TPRH, TPRH)],
                  out_hbm.at[cid, pl.ds(sid * TPRH, TPRH)])


def _sc_aggregate(h, bsrc, bdst, ncnt):
  kern = pl.kernel(
      _agg_body,
      out_type=jax.ShapeDtypeStruct((NC, ACC_H, D), jnp.float32),
      mesh=_sc_mesh(),
      scratch_types=[
          pltpu.VMEM((ASLAB, CHUNK), jnp.int32),
          pltpu.VMEM((ASLAB, CHUNK), jnp.int32),
          pltpu.VMEM((CHUNK, D), jnp.float32),
          pltpu.VMEM((CHUNK, D), jnp.float32),
          pltpu.SMEM((64,), jnp.int32),
          pltpu.VMEM_SHARED((NW, CHUNK), jnp.int32),
          pltpu.VMEM_SHARED((HN, D), jnp.float32),
          pltpu.VMEM_SHARED((ACC_H, D), jnp.float32),
          pltpu.SemaphoreType.DMA,
          pltpu.SemaphoreType.DMA,
      ],
  )
  return kern(h, bsrc, bdst, ncnt)


def _mm_body(x_ref, w_ref, o_ref):
  o_ref[...] = jnp.dot(x_ref[...], w_ref[...],
                       preferred_element_type=jnp.float32)


def _tc_matmul(x, w):
  return pl.pallas_call(
      _mm_body,
      grid=(N // ROW_BLK,),
      in_specs=[
          pl.BlockSpec((ROW_BLK, D), lambda i: (i, 0)),
          pl.BlockSpec((D, D), lambda i: (0, 0)),
      ],
      out_specs=pl.BlockSpec((ROW_BLK, D), lambda i: (i, 0)),
      out_shape=jax.ShapeDtypeStruct((N, D), jnp.float32),
  )(x, w)


_NBLK_H = HN // ROW_BLK


def _p0_map(i):
  return (jnp.where(i < _NBLK_H, i, 0), 0)


def _p1_map(i):
  return (jnp.where(i < _NBLK_H, 0, i - _NBLK_H), 0)


def _comb_body(p0_ref, p1_ref, x_ref, w_ref, xn_ref, h_ref):
  agg = jnp.where(pl.program_id(0) < _NBLK_H, p0_ref[...], p1_ref[...])
  xn = x_ref[...] + jnp.maximum(agg, 0.0)
  xn_ref[...] = xn
  h_ref[...] = jnp.dot(xn, w_ref[...], preferred_element_type=jnp.float32)


def _tc_combine(p, x, w):
  hblk0 = pl.BlockSpec((ROW_BLK, D), _p0_map)
  hblk1 = pl.BlockSpec((ROW_BLK, D), _p1_map)
  blk = pl.BlockSpec((ROW_BLK, D), lambda i: (i, 0))
  return pl.pallas_call(
      _comb_body,
      grid=(N // ROW_BLK,),
      in_specs=[hblk0, hblk1, blk, pl.BlockSpec((D, D), lambda i: (0, 0))],
      out_specs=[blk, blk],
      out_shape=[
          jax.ShapeDtypeStruct((N, D), jnp.float32),
          jax.ShapeDtypeStruct((N, D), jnp.float32),
      ],
  )(p[0], p[1], x, w)


def _relu_body(p0_ref, p1_ref, o_ref):
  agg = jnp.where(pl.program_id(0) < _NBLK_H, p0_ref[...], p1_ref[...])
  o_ref[...] = jnp.maximum(agg, 0.0)


def _tc_final(p):
  hblk0 = pl.BlockSpec((ROW_BLK, D), _p0_map)
  hblk1 = pl.BlockSpec((ROW_BLK, D), _p1_map)
  return pl.pallas_call(
      _relu_body,
      grid=(N // ROW_BLK,),
      in_specs=[hblk0, hblk1],
      out_specs=pl.BlockSpec((ROW_BLK, D), lambda i: (i, 0)),
      out_shape=jax.ShapeDtypeStruct((N, D), jnp.float32),
  )(p[0], p[1])


@jax.jit
def kernel(x, edge_index, W1, W2, W3):
  src = edge_index[0].astype(jnp.int32)
  dst = edge_index[1].astype(jnp.int32)
  ept = E // NW
  ppt = ETW - ept
  pad_src = jnp.zeros((NW, ppt), jnp.int32)
  pad_dst = jnp.broadcast_to(
      N + (jnp.arange(ppt, dtype=jnp.int32) % SPARE), (NW, ppt))
  src_b = jnp.concatenate(
      [src.reshape(NW, ept), pad_src], axis=1).reshape(NW, BROWS, CHUNK)
  dst_b = jnp.concatenate(
      [dst.reshape(NW, ept), pad_dst], axis=1).reshape(NW, BROWS, CHUNK)

  bsrc, bdst, ncnt = _sc_bin(src_b, dst_b)

  h1 = _tc_matmul(x, W1)
  p1 = _sc_aggregate(h1, bsrc, bdst, ncnt)
  x2, h2 = _tc_combine(p1, x, W2)
  p2 = _sc_aggregate(h2, bsrc, bdst, ncnt)
  x3, h3 = _tc_combine(p2, x2, W3)
  p3 = _sc_aggregate(h3, bsrc, bdst, ncnt)
  return _tc_final(p3)


# speedup vs baseline: 1.4312x; 1.4312x over previous
import jax
import jax.numpy as jnp
from jax import lax
from jax.experimental import pallas as pl
from jax.experimental.pallas import tpu as pltpu
from jax.experimental.pallas import tpu_sc as plsc

N = 10000
D = 128
E = 320000

NC = 2
NS = 16
NW = NC * NS

CHUNK = 128
NCH = 80
HALF = NCH // 2
E_PAD = NW * NCH * CHUNK
TPR = 632
ACC_N = NS * TPR
DUMMY = N

ROW_BLK = 2000


def _sc_mesh():
  return plsc.VectorSubcoreMesh(core_axis_name="c", subcore_axis_name="s")


def _sc_body(h_hbm, src_hbm, dst_hbm, out_hbm,
             src_v, dst_v, rows_a, rows_b, acc_sh, sem_a, sem_b):
  cid = lax.axis_index("c")
  sid = lax.axis_index("s")
  wid = cid * NS + sid

  def _zrow(i, carry):
    for j in range(D // 16):
      rows_a[i, pl.ds(j * 16, 16)] = jnp.zeros((16,), jnp.float32)
    return carry
  lax.fori_loop(0, CHUNK, _zrow, 0)
  zbase = sid * TPR
  nfull = TPR // CHUNK
  for k in range(nfull):
    pltpu.sync_copy(rows_a, acc_sh.at[pl.ds(zbase + k * CHUNK, CHUNK)])
  rem = TPR - nfull * CHUNK
  if rem:
    pltpu.sync_copy(rows_a.at[pl.ds(0, rem)],
                    acc_sh.at[pl.ds(zbase + nfull * CHUNK, rem)])
  plsc.subcore_barrier()

  def _start(j, buf, sem):
    pltpu.async_copy(h_hbm.at[src_v.at[j]], buf, sem)

  def _wait(j, buf, sem):
    pltpu.make_async_copy(h_hbm.at[src_v.at[j]], buf, sem).wait()

  def _scat(j, buf):
    pltpu.sync_copy(buf, acc_sh.at[dst_v.at[j]], add=True)

  def _pipe(i, carry):
    j = i * 2
    _start(j + 1, rows_b, sem_b)
    _wait(j, rows_a, sem_a)
    _scat(j, rows_a)

    @pl.when(j + 2 < HALF)
    def _():
      _start(j + 2, rows_a, sem_a)

    _wait(j + 1, rows_b, sem_b)
    _scat(j + 1, rows_b)
    return carry

  for half in range(NCH // HALF):
    pltpu.sync_copy(src_hbm.at[wid, pl.ds(half * HALF, HALF)], src_v)
    pltpu.sync_copy(dst_hbm.at[wid, pl.ds(half * HALF, HALF)], dst_v)
    _start(0, rows_a, sem_a)
    lax.fori_loop(0, HALF // 2, _pipe, 0)
  plsc.subcore_barrier()

  pltpu.sync_copy(acc_sh.at[pl.ds(sid * TPR, TPR)],
                  out_hbm.at[cid, pl.ds(sid * TPR, TPR)])


def _sc_aggregate(h, src_r, dst_r):
  kern = pl.kernel(
      _sc_body,
      out_type=jax.ShapeDtypeStruct((NC, ACC_N, D), jnp.float32),
      mesh=_sc_mesh(),
      scratch_types=[
          pltpu.VMEM((HALF, CHUNK), jnp.int32),
          pltpu.VMEM((HALF, CHUNK), jnp.int32),
          pltpu.VMEM((CHUNK, D), jnp.float32),
          pltpu.VMEM((CHUNK, D), jnp.float32),
          pltpu.VMEM_SHARED((ACC_N, D), jnp.float32),
          pltpu.SemaphoreType.DMA,
          pltpu.SemaphoreType.DMA,
      ],
  )
  return kern(h, src_r, dst_r)


def _mm_body(x_ref, w_ref, o_ref):
  o_ref[...] = jnp.dot(x_ref[...], w_ref[...],
                       preferred_element_type=jnp.float32)


def _tc_matmul(x, w):
  return pl.pallas_call(
      _mm_body,
      grid=(N // ROW_BLK,),
      in_specs=[
          pl.BlockSpec((ROW_BLK, D), lambda i: (i, 0)),
          pl.BlockSpec((D, D), lambda i: (0, 0)),
      ],
      out_specs=pl.BlockSpec((ROW_BLK, D), lambda i: (i, 0)),
      out_shape=jax.ShapeDtypeStruct((N, D), jnp.float32),
  )(x, w)


def _comb_body(p0_ref, p1_ref, x_ref, w_ref, xn_ref, h_ref):
  xn = x_ref[...] + jnp.maximum(p0_ref[...] + p1_ref[...], 0.0)
  xn_ref[...] = xn
  h_ref[...] = jnp.dot(xn, w_ref[...], preferred_element_type=jnp.float32)


def _tc_combine(p, x, w):
  blk = pl.BlockSpec((ROW_BLK, D), lambda i: (i, 0))
  return pl.pallas_call(
      _comb_body,
      grid=(N // ROW_BLK,),
      in_specs=[blk, blk, blk, pl.BlockSpec((D, D), lambda i: (0, 0))],
      out_specs=[blk, blk],
      out_shape=[
          jax.ShapeDtypeStruct((N, D), jnp.float32),
          jax.ShapeDtypeStruct((N, D), jnp.float32),
      ],
  )(p[0], p[1], x, w)


def _relu_body(p0_ref, p1_ref, o_ref):
  o_ref[...] = jnp.maximum(p0_ref[...] + p1_ref[...], 0.0)


def _tc_final(p):
  blk = pl.BlockSpec((ROW_BLK, D), lambda i: (i, 0))
  return pl.pallas_call(
      _relu_body,
      grid=(N // ROW_BLK,),
      in_specs=[blk, blk],
      out_specs=blk,
      out_shape=jax.ShapeDtypeStruct((N, D), jnp.float32),
  )(p[0], p[1])


@jax.jit
def kernel(x, edge_index, W1, W2, W3):
  src = edge_index[0].astype(jnp.int32)
  dst = edge_index[1].astype(jnp.int32)
  ept = E // NW
  ppt = NCH * CHUNK - ept
  pad_src = jnp.zeros((NW, ppt), jnp.int32)
  pad_dst = jnp.broadcast_to(
      DUMMY + (jnp.arange(ppt, dtype=jnp.int32) % (ACC_N - N)), (NW, ppt))
  src_r = jnp.concatenate(
      [src.reshape(NW, ept), pad_src], axis=1).reshape(NW, NCH, CHUNK)
  dst_r = jnp.concatenate(
      [dst.reshape(NW, ept), pad_dst], axis=1).reshape(NW, NCH, CHUNK)

  h1 = _tc_matmul(x, W1)
  p1 = _sc_aggregate(h1, src_r, dst_r)
  x2, h2 = _tc_combine(p1, x, W2)
  p2 = _sc_aggregate(h2, src_r, dst_r)
  x3, h3 = _tc_combine(p2, x2, W3)
  p3 = _sc_aggregate(h3, src_r, dst_r)
  return _tc_final(p3)
